# baseline (speedup 1.0000x reference)
import functools

import jax
import jax.numpy as jnp
from jax import lax
from jax.experimental import pallas as pl
from jax.experimental.pallas import tpu as pltpu
from jax.experimental.pallas import tpu_sc as plsc

VOCAB = 1000000
NUM_SPEAKERS = 100000
EMBED_DIM = 64
CONTEXT = 200
HIDDEN = 128
IN1 = EMBED_DIM + CONTEXT * EMBED_DIM

_NC = 2
_NS = 16
_CHUNK = 8
_NCHUNKS = CONTEXT // _CHUNK


def _sc_gather_body(speaker_hbm, widx_hbm, wtab_hbm, stab_hbm,
                    words_hbm, spk_hbm, idx_v, rows_v, sidx_v, srow_v, sem):
    wid = lax.axis_index("s") * _NC + lax.axis_index("c")

    @pl.when(wid < _NCHUNKS)
    def _words():
        base = wid * _CHUNK
        pltpu.sync_copy(widx_hbm.at[pl.ds(base, _CHUNK)], idx_v)
        pltpu.async_copy(wtab_hbm.at[idx_v], rows_v, sem).wait()
        pltpu.sync_copy(rows_v, words_hbm.at[pl.ds(base, _CHUNK)])

    @pl.when(wid == _NCHUNKS)
    def _speaker():
        pltpu.sync_copy(speaker_hbm, sidx_v)
        pltpu.async_copy(stab_hbm.at[sidx_v], srow_v, sem).wait()
        pltpu.sync_copy(srow_v, spk_hbm)


@functools.partial(
    pl.kernel,
    out_type=(jax.ShapeDtypeStruct((CONTEXT, EMBED_DIM), jnp.float32),
              jax.ShapeDtypeStruct((1, EMBED_DIM), jnp.float32)),
    mesh=plsc.VectorSubcoreMesh(core_axis_name="c", subcore_axis_name="s"),
    compiler_params=pltpu.CompilerParams(use_tc_tiling_on_sc=False),
    scratch_types=[
        pltpu.VMEM((_CHUNK,), jnp.int32),
        pltpu.VMEM((_CHUNK, EMBED_DIM), jnp.float32),
        pltpu.VMEM((1,), jnp.int32),
        pltpu.VMEM((1, EMBED_DIM), jnp.float32),
        pltpu.SemaphoreType.DMA,
    ],
)
def _sc_gather(*refs):
    _sc_gather_body(*refs)


_K = 8
_C = IN1 // _K


def _mlp_body(x_ref, w1_ref, b1_ref, w2_ref, b2_ref, out_ref, acc_ref):
    i = pl.program_id(0)

    @pl.when(i == 0)
    def _():
        acc_ref[...] = jnp.zeros_like(acc_ref)

    acc_ref[...] += jax.lax.dot_general(
        x_ref[...], w1_ref[...],
        dimension_numbers=(((0,), (0,)), ((), ())),
        preferred_element_type=jnp.float32)

    @pl.when(i == _K - 1)
    def _():
        h = jnp.maximum(acc_ref[...] + b1_ref[...], 0.0)
        o = jnp.dot(h, w2_ref[...], preferred_element_type=jnp.float32)
        out_ref[...] = jax.nn.sigmoid(o + b2_ref[...])


def _mlp(x, W1, b1, W2, b2):
    return pl.pallas_call(
        _mlp_body,
        grid=(_K,),
        in_specs=[
            pl.BlockSpec((_C, 1), lambda i: (i, 0)),
            pl.BlockSpec((_C, HIDDEN), lambda i: (i, 0)),
            pl.BlockSpec((1, HIDDEN), lambda i: (0, 0)),
            pl.BlockSpec((HIDDEN, 1), lambda i: (0, 0)),
            pl.BlockSpec((1, 1), lambda i: (0, 0)),
        ],
        out_specs=pl.BlockSpec((1, 1), lambda i: (0, 0)),
        out_shape=jax.ShapeDtypeStruct((1, 1), jnp.float32),
        scratch_shapes=[pltpu.VMEM((1, HIDDEN), jnp.float32)],
    )(x, W1, b1.reshape(1, HIDDEN), W2, b2.reshape(1, 1))


def kernel(speaker_code, word_indices, word_table, speaker_table, W1, b1, W2, b2):
    words, spk = _sc_gather(speaker_code.astype(jnp.int32),
                            word_indices.astype(jnp.int32),
                            word_table, speaker_table)
    x = jnp.concatenate([spk.reshape(-1), words.reshape(-1)]).reshape(IN1, 1)
    return _mlp(x, W1, b1, W2, b2)


# speedup vs baseline: 1.0145x; 1.0145x over previous
import functools

import jax
import jax.numpy as jnp
from jax import lax
from jax.experimental import pallas as pl
from jax.experimental.pallas import tpu as pltpu
from jax.experimental.pallas import tpu_sc as plsc

VOCAB = 1000000
NUM_SPEAKERS = 100000
EMBED_DIM = 64
CONTEXT = 200
HIDDEN = 128
IN1 = EMBED_DIM + CONTEXT * EMBED_DIM

_NC = 2
_NS = 16
_CHUNK = 8
_NCHUNKS = CONTEXT // _CHUNK


def _sc_gather_body(speaker_hbm, widx_hbm, wtab_hbm, stab_hbm,
                    words_hbm, spk_hbm, idx_v, rows_v, sidx_v, srow_v, sem):
    wid = lax.axis_index("s") * _NC + lax.axis_index("c")

    @pl.when(wid < _NCHUNKS)
    def _words():
        base = wid * _CHUNK
        pltpu.sync_copy(widx_hbm.at[pl.ds(base, _CHUNK)], idx_v)
        pltpu.async_copy(wtab_hbm.at[idx_v], rows_v, sem).wait()
        pltpu.sync_copy(rows_v, words_hbm.at[pl.ds(base, _CHUNK)])

    @pl.when(wid == _NCHUNKS)
    def _speaker():
        pltpu.sync_copy(speaker_hbm, sidx_v)
        pltpu.async_copy(stab_hbm.at[sidx_v], srow_v, sem).wait()
        pltpu.sync_copy(srow_v, spk_hbm)


@functools.partial(
    pl.kernel,
    out_type=(jax.ShapeDtypeStruct((CONTEXT, EMBED_DIM), jnp.float32),
              jax.ShapeDtypeStruct((1, EMBED_DIM), jnp.float32)),
    mesh=plsc.VectorSubcoreMesh(core_axis_name="c", subcore_axis_name="s"),
    compiler_params=pltpu.CompilerParams(use_tc_tiling_on_sc=False),
    scratch_types=[
        pltpu.VMEM((_CHUNK,), jnp.int32),
        pltpu.VMEM((_CHUNK, EMBED_DIM), jnp.float32),
        pltpu.VMEM((1,), jnp.int32),
        pltpu.VMEM((1, EMBED_DIM), jnp.float32),
        pltpu.SemaphoreType.DMA,
    ],
)
def _sc_gather(*refs):
    _sc_gather_body(*refs)


def _mlp_body(spk_ref, words_ref, w1_ref, b1_ref, w2_ref, b2_ref, out_ref):
    h = jnp.dot(spk_ref[...], w1_ref[0], preferred_element_type=jnp.float32)
    ww = w1_ref[pl.ds(1, CONTEXT)]
    hw = jnp.sum(words_ref[...][:, :, None] * ww, axis=(0, 1))
    h = jnp.maximum(h + hw[None, :] + b1_ref[...], 0.0)
    o = jnp.dot(h, w2_ref[...], preferred_element_type=jnp.float32)
    out_ref[...] = jax.nn.sigmoid(o + b2_ref[...])


def _mlp(spk, words, W1, b1, W2, b2):
    return pl.pallas_call(
        _mlp_body,
        out_shape=jax.ShapeDtypeStruct((1, 1), jnp.float32),
    )(spk, words, W1.reshape(CONTEXT + 1, EMBED_DIM, HIDDEN),
      b1.reshape(1, HIDDEN), W2, b2.reshape(1, 1))


def kernel(speaker_code, word_indices, word_table, speaker_table, W1, b1, W2, b2):
    words, spk = _sc_gather(speaker_code.astype(jnp.int32),
                            word_indices.astype(jnp.int32),
                            word_table, speaker_table)
    return _mlp(spk, words, W1, b1, W2, b2)


# speedup vs baseline: 1.6511x; 1.6274x over previous
import functools

import jax
import jax.numpy as jnp
from jax import lax
from jax.experimental import pallas as pl
from jax.experimental.pallas import tpu as pltpu

VOCAB = 1000000
NUM_SPEAKERS = 100000
EMBED_DIM = 64
CONTEXT = 200
HIDDEN = 128
NROWS = CONTEXT + 1
IN1 = NROWS * EMBED_DIM


def _fused_body(widx_ref, spk_ref, wtab_ref, stab_ref, w1_ref, b1_ref,
                w2_ref, b2_ref, out_ref, rows_v, sem):
    pltpu.make_async_copy(
        stab_ref.at[pl.ds(spk_ref[0], 1)], rows_v.at[pl.ds(0, 1)], sem
    ).start()

    def fire(i, carry):
        pltpu.make_async_copy(
            wtab_ref.at[pl.ds(widx_ref[i], 1)],
            rows_v.at[pl.ds(i + 1, 1)],
            sem,
        ).start()
        return carry

    lax.fori_loop(0, CONTEXT, fire, 0)

    pltpu.make_async_copy(wtab_ref.at[pl.ds(0, NROWS)], rows_v, sem).wait()

    def accum(c, h):
        x_c = rows_v[pl.ds(c, 1)]
        w_c = w1_ref[pl.ds(c, 1)][0]
        return h + jnp.dot(x_c, w_c, preferred_element_type=jnp.float32)

    h = lax.fori_loop(0, NROWS, accum, jnp.zeros((1, HIDDEN), jnp.float32))
    h = jnp.maximum(h + b1_ref[...], 0.0)
    o = jnp.dot(h, w2_ref[...], preferred_element_type=jnp.float32)
    out_ref[...] = jax.nn.sigmoid(o + b2_ref[...])


@functools.partial(jax.jit, static_argnames=())
def kernel(speaker_code, word_indices, word_table, speaker_table, W1, b1, W2, b2):
    grid_spec = pltpu.PrefetchScalarGridSpec(
        num_scalar_prefetch=2,
        grid=(1,),
        in_specs=[
            pl.BlockSpec(memory_space=pl.ANY),
            pl.BlockSpec(memory_space=pl.ANY),
            pl.BlockSpec((NROWS, EMBED_DIM, HIDDEN), lambda i, *_: (0, 0, 0)),
            pl.BlockSpec((1, HIDDEN), lambda i, *_: (0, 0)),
            pl.BlockSpec((HIDDEN, 1), lambda i, *_: (0, 0)),
            pl.BlockSpec((1, 1), lambda i, *_: (0, 0)),
        ],
        out_specs=pl.BlockSpec((1, 1), lambda i, *_: (0, 0)),
        scratch_shapes=[
            pltpu.VMEM((NROWS, EMBED_DIM), jnp.float32),
            pltpu.SemaphoreType.DMA,
        ],
    )
    return pl.pallas_call(
        _fused_body,
        grid_spec=grid_spec,
        out_shape=jax.ShapeDtypeStruct((1, 1), jnp.float32),
    )(word_indices.astype(jnp.int32), speaker_code.astype(jnp.int32),
      word_table, speaker_table, W1.reshape(NROWS, EMBED_DIM, HIDDEN),
      b1.reshape(1, HIDDEN), W2, b2.reshape(1, 1))
